# baseline (speedup 1.0000x reference)
import jax
import jax.numpy as jnp
import numpy as np
from jax.experimental import pallas as pl
from jax.experimental.pallas import tpu as pltpu

_BN_EPS = 1e-3


def _deepset_body(x_ref, w1_ref, g1_ref, b1_ref, w2_ref, g2_ref, b2_ref,
                  w3_ref, g3_ref, b3_ref, out_ref,
                  w1s_ref, w2s_ref, w3s_ref, b12s_ref,
                  hmc_ref, agg_ref, cnt_ref):
    inv = np.float32(1.0 / np.sqrt(1.0 + _BN_EPS))

    @pl.when(pl.program_id(0) == 0)
    def _prep():
        w1s_ref[...] = (w1_ref[...] * (g1_ref[...] * inv)[None, :]
                        ).astype(jnp.bfloat16)
        w2s_ref[...] = (w2_ref[...] * (g2_ref[...] * inv)[None, :]
                        ).astype(jnp.bfloat16)
        w3s_ref[...] = (w3_ref[...] * (g3_ref[...] * inv)[None, :]
                        ).astype(jnp.bfloat16)
        b12s_ref[...] = jnp.stack(
            [b1_ref[...], b2_ref[...]], axis=0).astype(jnp.bfloat16)

    L = x_ref.shape[2]
    n_chunks = 4
    C = L // n_chunks
    R = x_ref.shape[0]
    b1 = b12s_ref[0:1]
    b2 = b12s_ref[1:2]
    b3 = b3_ref[...][None, :]
    acc = [dict(p_sum=[], p_max=[], p_sq=[], p_cnt=[]) for _ in range(R)]

    def stage0(it):
        r, c = it
        xt = x_ref[r, :, c * C:(c + 1) * C]
        xc = jnp.transpose(xt)
        mask = jnp.any(xc != 0.0, axis=1, keepdims=True)
        x = xt.astype(jnp.bfloat16)
        h = jax.lax.dot_general(x, w1s_ref[...], (((0,), (0,)), ((), ())),
                                preferred_element_type=jnp.float32)
        h = jnp.maximum(h.astype(jnp.bfloat16) + b1, 0.0)
        return mask, h

    def stage1(h):
        h = jnp.dot(h, w2s_ref[...], preferred_element_type=jnp.float32)
        return jnp.maximum(h.astype(jnp.bfloat16) + b2, 0.0)

    def stage2(mask, h):
        h = jnp.dot(h, w3s_ref[...], preferred_element_type=jnp.float32)
        return jnp.maximum((h + b3) * mask.astype(jnp.float32), 0.0)

    def stage3(it, mask, hm):
        a = acc[it[0]]
        a["p_cnt"].append(jnp.sum(mask.astype(jnp.float32)))
        a["p_sum"].append(jnp.sum(hm, axis=0, keepdims=True))

    def stage4(it, hm):
        a = acc[it[0]]
        a["p_max"].append(jnp.max(hm, axis=0, keepdims=True))
        a["p_sq"].append(jnp.sum(hm * hm, axis=0, keepdims=True))

    items = [(r, c) for c in range(n_chunks) for r in range(R)]
    n = len(items)
    n_defer = 2
    masks, h1, h2, hm = {}, {}, {}, {}

    step = pl.program_id(0)
    n_steps = pl.num_programs(0)

    @pl.when(step > 0)
    def _finish_prev():
        for j in range(n_defer):
            r = items[n - n_defer + j][0]
            hmp = hmc_ref[j]
            d_sum = jnp.sum(hmp, axis=0, keepdims=True)
            d_max = jnp.max(hmp, axis=0, keepdims=True)
            d_sq = jnp.sum(hmp * hmp, axis=0, keepdims=True)
            cnt = cnt_ref[r:r + 1, 0:1] + jnp.sum(hmc_ref[2 + j, :, 0:1])
            agg_sum = agg_ref[3 * r + 0:3 * r + 1] + d_sum
            agg_max = jnp.maximum(agg_ref[3 * r + 1:3 * r + 2], d_max)
            ex2 = (agg_ref[3 * r + 2:3 * r + 3] + d_sq) / cnt
            agg_mean = agg_sum / cnt
            var = ex2 - agg_mean * agg_mean
            agg_std = jnp.sqrt(jnp.maximum(var, 1e-12))
            b = (step - 1) * R + r
            row = jnp.concatenate([agg_sum, agg_max, agg_mean, agg_std],
                                  axis=1)
            out_ref[pl.ds(b, 1), :] = row

    def run(stage, i):
        if i < 0 or i >= n:
            return
        it = items[i]
        deferred = i >= n - n_defer
        if stage == 0:
            masks[it], h1[it] = stage0(it)
        elif stage == 1:
            h2[it] = stage1(h1[it])
        elif stage == 2:
            if deferred:
                hmc_ref[i - (n - n_defer)] = stage2(masks[it], h2[it])
            else:
                hm[it] = stage2(masks[it], h2[it])
        elif stage == 3:
            if not deferred:
                stage3(it, masks[it], hm[it])
        else:
            if not deferred:
                stage4(it, hm[it])

    for i in range(n + 4):
        run(0, i)
        run(1, i - 1)
        run(2, i - 2)
        run(3, i - 3)
        run(4, i - 4)

    part = {}
    for r in range(R):
        a = acc[r]
        pmax = a["p_max"][0]
        for pm in a["p_max"][1:]:
            pmax = jnp.maximum(pmax, pm)
        part[r] = (sum(a["p_cnt"]), sum(a["p_sum"]), pmax, sum(a["p_sq"]))

    @pl.when(step < n_steps - 1)
    def _stash():
        for j in range(n_defer):
            it = items[n - n_defer + j]
            hmc_ref[2 + j, :, 0:1] = masks[it].astype(jnp.float32)
        for r in range(R):
            cnt_r, sum_r, max_r, sq_r = part[r]
            cnt_ref[r:r + 1, 0:1] = cnt_r[None, None]
            agg_ref[3 * r + 0:3 * r + 1] = sum_r
            agg_ref[3 * r + 1:3 * r + 2] = max_r
            agg_ref[3 * r + 2:3 * r + 3] = sq_r

    @pl.when(step == n_steps - 1)
    def _finish_last():
        for j in range(n_defer):
            it = items[n - n_defer + j]
            r = it[0]
            cnt_r, sum_r, max_r, sq_r = part[r]
            hmp = hmc_ref[j]
            cnt = cnt_r + jnp.sum(masks[it].astype(jnp.float32))
            agg_sum = sum_r + jnp.sum(hmp, axis=0, keepdims=True)
            agg_max = jnp.maximum(max_r, jnp.max(hmp, axis=0, keepdims=True))
            ex2 = (sq_r + jnp.sum(hmp * hmp, axis=0, keepdims=True)) / cnt
            agg_mean = agg_sum / cnt
            var = ex2 - agg_mean * agg_mean
            agg_std = jnp.sqrt(jnp.maximum(var, 1e-12))
            b = step * R + r
            row = jnp.concatenate([agg_sum, agg_max, agg_mean, agg_std],
                                  axis=1)
            out_ref[pl.ds(b, 1), :] = row


def kernel(inputs, W1, g1, b1, W2, g2, b2, W3, g3, b3):
    B, L, F = inputs.shape
    H = W3.shape[1]

    full = lambda shape: pl.BlockSpec(shape, lambda b: (0,) * len(shape))
    return pl.pallas_call(
        _deepset_body,
        grid=(B // 2,),
        in_specs=[
            pl.BlockSpec((2, F, L), lambda b: (b, 0, 0)),
            full(W1.shape), full(g1.shape), full(b1.shape),
            full(W2.shape), full(g2.shape), full(b2.shape),
            full(W3.shape), full(g3.shape), full(b3.shape),
        ],
        out_specs=pl.BlockSpec((B, 4 * H), lambda b: (0, 0)),
        out_shape=jax.ShapeDtypeStruct((B, 4 * H), jnp.float32),
        scratch_shapes=[
            pltpu.VMEM((F, H), jnp.bfloat16),
            pltpu.VMEM((H, H), jnp.bfloat16),
            pltpu.VMEM((H, H), jnp.bfloat16),
            pltpu.VMEM((2, H), jnp.bfloat16),
            pltpu.VMEM((4, L // 4, H), jnp.float32),
            pltpu.VMEM((6, H), jnp.float32),
            pltpu.VMEM((8, 128), jnp.float32),
        ],
        compiler_params=pltpu.CompilerParams(
            dimension_semantics=("arbitrary",)),
    )(inputs.transpose(0, 2, 1), W1, g1, b1, W2, g2, b2, W3, g3, b3)


# speedup vs baseline: 1.0470x; 1.0470x over previous
import jax
import jax.numpy as jnp
import numpy as np
from jax.experimental import pallas as pl
from jax.experimental.pallas import tpu as pltpu

_BN_EPS = 1e-3


def _deepset_body(x_ref, w1_ref, g1_ref, b1_ref, w2_ref, g2_ref, b2_ref,
                  w3_ref, g3_ref, b3_ref, out_ref,
                  w1s_ref, w2s_ref, w3s_ref, b12s_ref):
    inv = np.float32(1.0 / np.sqrt(1.0 + _BN_EPS))

    @pl.when(pl.program_id(0) == 0)
    def _prep():
        w1s_ref[...] = (w1_ref[...] * (g1_ref[...] * inv)[None, :]
                        ).astype(jnp.bfloat16)
        w2s_ref[...] = (w2_ref[...] * (g2_ref[...] * inv)[None, :]
                        ).astype(jnp.bfloat16)
        w3s_ref[...] = (w3_ref[...] * (g3_ref[...] * inv)[None, :]
                        ).astype(jnp.bfloat16)
        b12s_ref[...] = jnp.stack(
            [b1_ref[...], b2_ref[...]], axis=0).astype(jnp.bfloat16)

    L = x_ref.shape[2]
    n_chunks = 4
    C = L // n_chunks
    R = x_ref.shape[0]
    b1 = b12s_ref[0:1]
    b2 = b12s_ref[1:2]
    b3 = b3_ref[...][None, :]
    acc = [dict(p_sum=[], p_max=[], p_sq=[], p_cnt=[]) for _ in range(R)]

    def stage0(it):
        r, c = it
        xt = x_ref[r, :, c * C:(c + 1) * C]
        xc = jnp.transpose(xt)
        mask = jnp.any(xc != 0.0, axis=1, keepdims=True)
        x = xt.astype(jnp.bfloat16)
        h = jax.lax.dot_general(x, w1s_ref[...], (((0,), (0,)), ((), ())),
                                preferred_element_type=jnp.float32)
        h = jnp.maximum(h.astype(jnp.bfloat16) + b1, 0.0)
        return mask, h

    def stage1(h):
        h = jnp.dot(h, w2s_ref[...], preferred_element_type=jnp.float32)
        return jnp.maximum(h.astype(jnp.bfloat16) + b2, 0.0)

    def stage2(mask, h):
        h = jnp.dot(h, w3s_ref[...], preferred_element_type=jnp.float32)
        return jnp.maximum((h + b3) * mask.astype(jnp.float32), 0.0)

    def stage3(it, mask, hm):
        a = acc[it[0]]
        a["p_cnt"].append(jnp.sum(mask.astype(jnp.float32)))
        a["p_sum"].append(jnp.sum(hm, axis=0, keepdims=True))

    def stage4(it, hm):
        a = acc[it[0]]
        a["p_max"].append(jnp.max(hm, axis=0, keepdims=True))
        a["p_sq"].append(jnp.sum(hm * hm, axis=0, keepdims=True))

    items = [(r, c) for c in range(n_chunks) for r in range(R)]
    n = len(items)
    masks, h1, h2, hm = {}, {}, {}, {}

    def run(stage, i):
        if i < 0 or i >= n:
            return
        it = items[i]
        if stage == 0:
            masks[it], h1[it] = stage0(it)
        elif stage == 1:
            h2[it] = stage1(h1[it])
        elif stage == 2:
            hm[it] = stage2(masks[it], h2[it])
        elif stage == 3:
            stage3(it, masks[it], hm[it])
        else:
            stage4(it, hm[it])

    for i in range(n + 4):
        run(0, i)
        run(1, i - 1)
        run(2, i - 2)
        run(3, i - 3)
        run(4, i - 4)

    for r in range(R):
        a = acc[r]
        cnt = sum(a["p_cnt"])
        agg_sum = sum(a["p_sum"])
        agg_max = a["p_max"][0]
        for pm in a["p_max"][1:]:
            agg_max = jnp.maximum(agg_max, pm)
        ex2 = sum(a["p_sq"]) / cnt
        agg_mean = agg_sum / cnt
        var = ex2 - agg_mean * agg_mean
        agg_std = jnp.sqrt(jnp.maximum(var, 1e-12))
        H = agg_sum.shape[1]
        b = pl.program_id(0) * R + r
        row = jnp.concatenate([agg_sum, agg_max, agg_mean, agg_std], axis=1)
        out_ref[pl.ds(b, 1), :] = row


def kernel(inputs, W1, g1, b1, W2, g2, b2, W3, g3, b3):
    B, L, F = inputs.shape
    H = W3.shape[1]

    full = lambda shape: pl.BlockSpec(shape, lambda b: (0,) * len(shape))
    return pl.pallas_call(
        _deepset_body,
        grid=(B // 2,),
        in_specs=[
            pl.BlockSpec((2, F, L), lambda b: (b, 0, 0)),
            full(W1.shape), full(g1.shape), full(b1.shape),
            full(W2.shape), full(g2.shape), full(b2.shape),
            full(W3.shape), full(g3.shape), full(b3.shape),
        ],
        out_specs=pl.BlockSpec((B, 4 * H), lambda b: (0, 0)),
        out_shape=jax.ShapeDtypeStruct((B, 4 * H), jnp.float32),
        scratch_shapes=[
            pltpu.VMEM((F, H), jnp.bfloat16),
            pltpu.VMEM((H, H), jnp.bfloat16),
            pltpu.VMEM((H, H), jnp.bfloat16),
            pltpu.VMEM((2, H), jnp.bfloat16),
        ],
        compiler_params=pltpu.CompilerParams(
            dimension_semantics=("arbitrary",)),
    )(inputs.transpose(0, 2, 1), W1, g1, b1, W2, g2, b2, W3, g3, b3)
